# baseline (speedup 1.0000x reference)
import functools

import jax
import jax.numpy as jnp
from jax import lax
from jax.experimental import pallas as pl
from jax.experimental.pallas import tpu as pltpu
from jax.experimental.pallas import tpu_sc as plsc

NUM_CORES = 2
NUM_SUBCORES = 16
NUM_WORKERS = NUM_CORES * NUM_SUBCORES
B = 16384
EMB = 64
CHUNK = 128
B_PER_W = B // NUM_WORKERS
N_CHUNKS = B_PER_W // CHUNK


def _gather3_body(users_hbm, items_hbm, neg_hbm, u_tab, v_tab,
                  out_u, out_v, out_n,
                  idx_u, idx_v, idx_n, rows_u, rows_v, rows_n,
                  idx_sem, gat_sem, out_sem):
    wid = lax.axis_index("s") * NUM_CORES + lax.axis_index("c")
    base = wid * B_PER_W

    idx_copies = []
    for j in range(N_CHUNKS):
        off = base + j * CHUNK
        for src, dst in ((users_hbm, idx_u), (items_hbm, idx_v),
                         (neg_hbm, idx_n)):
            c = pltpu.make_async_copy(src.at[pl.ds(off, CHUNK)],
                                      dst.at[j], idx_sem)
            c.start()
            idx_copies.append(c)
    for c in idx_copies:
        c.wait()

    gathers = []
    for j in range(N_CHUNKS):
        for tab, idx, rows in ((u_tab, idx_u, rows_u),
                               (v_tab, idx_v, rows_v),
                               (v_tab, idx_n, rows_n)):
            c = pltpu.make_async_copy(tab.at[idx.at[j]], rows.at[j],
                                      gat_sem)
            c.start()
            gathers.append(c)
    for c in gathers:
        c.wait()

    outs = []
    for j in range(N_CHUNKS):
        off = base + j * CHUNK
        for rows, dst in ((rows_u, out_u), (rows_v, out_v),
                          (rows_n, out_n)):
            c = pltpu.make_async_copy(rows.at[j], dst.at[pl.ds(off, CHUNK)],
                                      out_sem)
            c.start()
            outs.append(c)
    for c in outs:
        c.wait()


@jax.jit
def kernel(users, items, neg_items, U, V):
    mesh = plsc.VectorSubcoreMesh(core_axis_name="c", subcore_axis_name="s",
                                  num_cores=NUM_CORES,
                                  num_subcores=NUM_SUBCORES)
    out_sd = jax.ShapeDtypeStruct((B, EMB), jnp.float32)
    f = pl.kernel(
        _gather3_body,
        out_type=(out_sd, out_sd, out_sd),
        mesh=mesh,
        compiler_params=pltpu.CompilerParams(use_tc_tiling_on_sc=False),
        scratch_types=[
            pltpu.VMEM((N_CHUNKS, CHUNK), jnp.int32),
            pltpu.VMEM((N_CHUNKS, CHUNK), jnp.int32),
            pltpu.VMEM((N_CHUNKS, CHUNK), jnp.int32),
            pltpu.VMEM((N_CHUNKS, CHUNK, EMB), jnp.float32),
            pltpu.VMEM((N_CHUNKS, CHUNK, EMB), jnp.float32),
            pltpu.VMEM((N_CHUNKS, CHUNK, EMB), jnp.float32),
            pltpu.SemaphoreType.DMA,
            pltpu.SemaphoreType.DMA,
            pltpu.SemaphoreType.DMA,
        ],
    )
    return f(users, items, neg_items, U, V)
